# baseline (speedup 1.0000x reference)
import functools

import jax
import jax.numpy as jnp
from jax import lax
from jax.experimental import pallas as pl
from jax.experimental.pallas import tpu as pltpu
from jax.experimental.pallas import tpu_sc as plsc

N = 10000
E = 320000
D = 128

NUM_TILES = 16
LANES = 128
EH = E // 2
EPT = 80
HALF_ROWS = EPT * NUM_TILES
IDX_ROWS = 2 * HALF_ROWS
EPT_DEG = IDX_ROWS // NUM_TILES
N_ACC = 10240
ROWS_PER_TILE = N_ACC // NUM_TILES


def _sc_mesh():
  return plsc.VectorSubcoreMesh(core_axis_name="c", subcore_axis_name="s")


def _deg_body(dst1, dst2, zeros1d, deg1, deg2, dstv, ones_v, acc, sem):
  cid = lax.axis_index("c")
  sid = lax.axis_index("s")
  for c in range(8):
    ones_v[pl.ds(c * 16, 16)] = jnp.ones((16,), jnp.float32)
  pltpu.async_copy(zeros1d, acc.at[pl.ds(sid * ROWS_PER_TILE, ROWS_PER_TILE)],
                   sem).wait()
  plsc.subcore_barrier()

  def run(dstm):
    pltpu.async_copy(dstm.at[pl.ds(sid * EPT_DEG, EPT_DEG)], dstv, sem).wait()

    def body(j, carry):
      pltpu.async_copy(ones_v, acc.at[dstv.at[j]], sem, add=True).wait()
      return carry

    lax.fori_loop(0, EPT_DEG, body, 0)

  @pl.when(cid == 0)
  def _():
    run(dst1)

  @pl.when(cid == 1)
  def _():
    run(dst2)

  plsc.subcore_barrier()

  def wb(out):
    pltpu.async_copy(acc.at[pl.ds(sid * ROWS_PER_TILE, ROWS_PER_TILE)],
                     out.at[pl.ds(sid * ROWS_PER_TILE, ROWS_PER_TILE)],
                     sem).wait()

  @pl.when(cid == 0)
  def _():
    wb(deg1)

  @pl.when(cid == 1)
  def _():
    wb(deg2)


def _make_deg_kernel():
  return functools.partial(
      pl.kernel,
      mesh=_sc_mesh(),
      out_type=(jax.ShapeDtypeStruct((N_ACC,), jnp.float32),
                jax.ShapeDtypeStruct((N_ACC,), jnp.float32)),
      scratch_types=[
          pltpu.VMEM((EPT_DEG, LANES), jnp.int32),
          pltpu.VMEM((LANES,), jnp.float32),
          pltpu.VMEM_SHARED((N_ACC,), jnp.float32),
          pltpu.SemaphoreType.DMA,
      ],
  )(_deg_body)


NBUF = 4
NGRP = EPT // NBUF - 1


def _spmm_body(src1, dst1, src2, dst2, t1b, t2b, zeros2d,
               s1a, s1b_, s2a, s2b_, srcv, dstv, rows, acc, *sems):
  cid = lax.axis_index("c")
  sid = lax.axis_index("s")
  my_rows = pl.ds(sid * ROWS_PER_TILE, ROWS_PER_TILE)
  sem = sems[0]
  gsem = sems[1:1 + NBUF]
  ssem = sems[1 + NBUF:1 + 2 * NBUF]
  idx_base = cid * HALF_ROWS + sid * EPT

  def one_view(srcm, dstm, t_hbm, out):
    pltpu.async_copy(zeros2d, acc.at[my_rows], sem).wait()
    pltpu.async_copy(srcm.at[pl.ds(idx_base, EPT)], srcv, sem).wait()
    pltpu.async_copy(dstm.at[pl.ds(idx_base, EPT)], dstv, sem).wait()
    plsc.subcore_barrier()

    def gather(j, b):
      pltpu.make_async_copy(t_hbm.at[srcv.at[j]], rows.at[b], gsem[b]).start()

    def gather_wait(j, b):
      pltpu.make_async_copy(t_hbm.at[srcv.at[j]], rows.at[b], gsem[b]).wait()

    def scatter(j, b):
      pltpu.make_async_copy(rows.at[b], acc.at[dstv.at[j]], ssem[b]).start(
          add=True)

    def scatter_wait(j, b):
      pltpu.make_async_copy(rows.at[b], acc.at[dstv.at[j]], ssem[b]).wait()

    for b in range(NBUF):
      gather(b, b)

    def body(g, carry):
      for b in range(NBUF):
        j = g * NBUF + b
        gather_wait(j, b)
        scatter(j, b)
      for b in range(NBUF):
        j = g * NBUF + b
        scatter_wait(j, b)
        gather(j + NBUF, b)
      return carry

    lax.fori_loop(0, NGRP, body, 0)
    for b in range(NBUF):
      j = NGRP * NBUF + b
      gather_wait(j, b)
      scatter(j, b)
    for b in range(NBUF):
      scatter_wait(NGRP * NBUF + b, b)

    plsc.subcore_barrier()
    pltpu.async_copy(acc.at[my_rows], out.at[my_rows], sem).wait()
    plsc.subcore_barrier()

  @pl.when(cid == 0)
  def _():
    one_view(src1, dst1, t1b, s1a)
    one_view(src2, dst2, t2b, s2a)

  @pl.when(cid == 1)
  def _():
    one_view(src1, dst1, t1b, s1b_)
    one_view(src2, dst2, t2b, s2b_)


def _make_spmm_kernel():
  part = jax.ShapeDtypeStruct((N_ACC, D), jnp.int16)
  return functools.partial(
      pl.kernel,
      mesh=_sc_mesh(),
      out_type=(part, part, part, part),
      compiler_params=pltpu.CompilerParams(use_tc_tiling_on_sc=False),
      scratch_types=[
          pltpu.VMEM((EPT, LANES), jnp.int32),
          pltpu.VMEM((EPT, LANES), jnp.int32),
          pltpu.VMEM((NBUF, LANES, D), jnp.int16),
          pltpu.VMEM_SHARED((N_ACC, D), jnp.int16),
      ] + [pltpu.SemaphoreType.DMA] * (1 + 2 * NBUF),
  )(_spmm_body)


def _dinv(deg_col):
  return lax.rsqrt(jnp.maximum(deg_col + 1.0, 1e-12))


CAP = 1024.0


def _store_t(f32_ref, q_ref, scale_ref, t):
  s = CAP / jnp.maximum(jnp.max(jnp.abs(t)), 1e-30)
  scale_ref[...] = jnp.full((1, 1), s, jnp.float32)
  f32_ref[pl.ds(0, N), :] = t
  f32_ref[pl.ds(N, 16), :] = jnp.zeros((16, D), jnp.float32)
  q_ref[pl.ds(0, N), :] = jnp.round(t * s).astype(jnp.int16)
  q_ref[pl.ds(N, 16), :] = jnp.zeros((16, D), jnp.int16)


def _tc_pre_body(x_ref, deg_ref, w_ref, f32_ref, q_ref, scale_ref):
  dinv = _dinv(deg_ref[...])
  t = dinv * jnp.dot(x_ref[...], w_ref[...],
                     preferred_element_type=jnp.float32)
  _store_t(f32_ref, q_ref, scale_ref, t)


def _tc_pre(x, deg_col, w):
  return pl.pallas_call(
      _tc_pre_body,
      out_shape=(jax.ShapeDtypeStruct((N_ACC, D), jnp.float32),
                 jax.ShapeDtypeStruct((N_ACC, D), jnp.int16),
                 jax.ShapeDtypeStruct((1, 1), jnp.float32)),
  )(x, deg_col, w)


def _combine(sa_ref, sb_ref, scale_ref):
  return (sa_ref[pl.ds(0, N), :].astype(jnp.float32) +
          sb_ref[pl.ds(0, N), :].astype(jnp.float32)) * (1.0 / scale_ref[...])


def _tc_mid_body(sa_ref, sb_ref, t_ref, scale_ref, deg_ref, b_ref, w_ref,
                 f32_ref, q_ref, scale2_ref):
  dinv = _dinv(deg_ref[...])
  h = jax.nn.relu(
      dinv * (_combine(sa_ref, sb_ref, scale_ref) + t_ref[pl.ds(0, N), :])
      + b_ref[...])
  t2 = dinv * jnp.dot(h, w_ref[...], preferred_element_type=jnp.float32)
  _store_t(f32_ref, q_ref, scale2_ref, t2)


def _tc_mid(sa, sb, t, scale, deg_col, b, w):
  return pl.pallas_call(
      _tc_mid_body,
      out_shape=(jax.ShapeDtypeStruct((N_ACC, D), jnp.float32),
                 jax.ShapeDtypeStruct((N_ACC, D), jnp.int16),
                 jax.ShapeDtypeStruct((1, 1), jnp.float32)),
  )(sa, sb, t, scale, deg_col, b, w)


def _tc_fin_body(sa_ref, sb_ref, t_ref, scale_ref, deg_ref, b_ref, out_ref):
  dinv = _dinv(deg_ref[...])
  h = (dinv * (_combine(sa_ref, sb_ref, scale_ref) + t_ref[pl.ds(0, N), :])
       + b_ref[...])
  mu = jnp.sum(h, axis=0, keepdims=True) * (1.0 / N)
  d = h - mu
  var = jnp.sum(d * d, axis=0, keepdims=True) * (1.0 / (N - 1))
  out_ref[...] = d / jnp.sqrt(var)


def _tc_fin(sa, sb, t, scale, deg_col, b):
  return pl.pallas_call(
      _tc_fin_body,
      out_shape=jax.ShapeDtypeStruct((N, D), jnp.float32),
  )(sa, sb, t, scale, deg_col, b)


def _prep_edges(ei):
  pad = HALF_ROWS * LANES - EH
  fill = jnp.full((pad,), N, jnp.int32)

  def prep(row):
    row = row.astype(jnp.int32)
    return jnp.concatenate([row[:EH], fill, row[EH:], fill]).reshape(
        IDX_ROWS, LANES)

  return prep(ei[0]), prep(ei[1])


def kernel(edge_index_1, x_1, edge_index_2, x_2, W1, b1, W2, b2):
  src1, dst1 = _prep_edges(edge_index_1)
  src2, dst2 = _prep_edges(edge_index_2)
  zeros1d = jnp.zeros((ROWS_PER_TILE,), jnp.float32)
  zeros2d = jnp.zeros((ROWS_PER_TILE, D), jnp.int16)
  b1r = b1.reshape(1, D)
  b2r = b2.reshape(1, D)

  deg_kernel = _make_deg_kernel()
  spmm_kernel = _make_spmm_kernel()

  deg1, deg2 = deg_kernel(dst1, dst2, zeros1d)
  deg1c = deg1[:N].reshape(N, 1)
  deg2c = deg2[:N].reshape(N, 1)

  t1_1, t1_1q, sc1_1 = _tc_pre(x_1, deg1c, W1)
  t1_2, t1_2q, sc1_2 = _tc_pre(x_2, deg2c, W1)

  s1_1a, s1_1b, s1_2a, s1_2b = spmm_kernel(
      src1, dst1, src2, dst2, t1_1q, t1_2q, zeros2d)

  t2_1, t2_1q, sc2_1 = _tc_mid(s1_1a, s1_1b, t1_1, sc1_1, deg1c, b1r, W2)
  t2_2, t2_2q, sc2_2 = _tc_mid(s1_2a, s1_2b, t1_2, sc1_2, deg2c, b1r, W2)

  s2_1a, s2_1b, s2_2a, s2_2b = spmm_kernel(
      src1, dst1, src2, dst2, t2_1q, t2_2q, zeros2d)

  z1 = _tc_fin(s2_1a, s2_1b, t2_1, sc2_1, deg1c, b2r)
  z2 = _tc_fin(s2_2a, s2_2b, t2_2, sc2_2, deg2c, b2r)
  return (z1, z2)


# speedup vs baseline: 1.0272x; 1.0272x over previous
import functools

import jax
import jax.numpy as jnp
from jax import lax
from jax.experimental import pallas as pl
from jax.experimental.pallas import tpu as pltpu
from jax.experimental.pallas import tpu_sc as plsc

N = 10000
E = 320000
D = 128

NUM_TILES = 16
LANES = 128
EH = E // 2
EPT = 80
HALF_ROWS = EPT * NUM_TILES
IDX_ROWS = 2 * HALF_ROWS
EPT_DEG = IDX_ROWS // NUM_TILES
N_ACC = 10240
ROWS_PER_TILE = N_ACC // NUM_TILES


def _sc_mesh():
  return plsc.VectorSubcoreMesh(core_axis_name="c", subcore_axis_name="s")


def _deg_body(dst1, dst2, zeros1d, deg1, deg2, dstv, ones_v, acc, sem):
  cid = lax.axis_index("c")
  sid = lax.axis_index("s")
  for c in range(8):
    ones_v[pl.ds(c * 16, 16)] = jnp.ones((16,), jnp.float32)
  pltpu.async_copy(zeros1d, acc.at[pl.ds(sid * ROWS_PER_TILE, ROWS_PER_TILE)],
                   sem).wait()
  plsc.subcore_barrier()

  def run(dstm):
    pltpu.async_copy(dstm.at[pl.ds(sid * EPT_DEG, EPT_DEG)], dstv, sem).wait()

    def body(j, carry):
      pltpu.async_copy(ones_v, acc.at[dstv.at[j]], sem, add=True).wait()
      return carry

    lax.fori_loop(0, EPT_DEG, body, 0)

  @pl.when(cid == 0)
  def _():
    run(dst1)

  @pl.when(cid == 1)
  def _():
    run(dst2)

  plsc.subcore_barrier()

  def wb(out):
    pltpu.async_copy(acc.at[pl.ds(sid * ROWS_PER_TILE, ROWS_PER_TILE)],
                     out.at[pl.ds(sid * ROWS_PER_TILE, ROWS_PER_TILE)],
                     sem).wait()

  @pl.when(cid == 0)
  def _():
    wb(deg1)

  @pl.when(cid == 1)
  def _():
    wb(deg2)


def _make_deg_kernel():
  return functools.partial(
      pl.kernel,
      mesh=_sc_mesh(),
      out_type=(jax.ShapeDtypeStruct((N_ACC,), jnp.float32),
                jax.ShapeDtypeStruct((N_ACC,), jnp.float32)),
      scratch_types=[
          pltpu.VMEM((EPT_DEG, LANES), jnp.int32),
          pltpu.VMEM((LANES,), jnp.float32),
          pltpu.VMEM_SHARED((N_ACC,), jnp.float32),
          pltpu.SemaphoreType.DMA,
      ],
  )(_deg_body)


NBUF = 4
NGRP = EPT // NBUF - 1


def _spmm_body(src1, dst1, src2, dst2, t1b, t2b, zeros2d,
               s1a, s1b_, s2a, s2b_, srcv, dstv, rows, acc, *sems):
  cid = lax.axis_index("c")
  sid = lax.axis_index("s")
  my_rows = pl.ds(sid * ROWS_PER_TILE, ROWS_PER_TILE)
  sem = sems[0]
  gsem = sems[1:1 + NBUF]
  ssem = sems[1 + NBUF:1 + 2 * NBUF]
  idx_base = cid * HALF_ROWS + sid * EPT

  def one_view(srcm, dstm, t_hbm, out):
    pltpu.async_copy(zeros2d, acc.at[my_rows], sem).wait()
    pltpu.async_copy(srcm.at[pl.ds(idx_base, EPT)], srcv, sem).wait()
    pltpu.async_copy(dstm.at[pl.ds(idx_base, EPT)], dstv, sem).wait()
    plsc.subcore_barrier()

    def gather(j, b):
      pltpu.make_async_copy(t_hbm.at[srcv.at[j]], rows.at[b], gsem[b]).start()

    def gather_wait(j, b):
      pltpu.make_async_copy(t_hbm.at[srcv.at[j]], rows.at[b], gsem[b]).wait()

    def scatter(j, b):
      pltpu.make_async_copy(rows.at[b], acc.at[dstv.at[j]], ssem[b]).start(
          add=True)

    def scatter_wait(j, b):
      pltpu.make_async_copy(rows.at[b], acc.at[dstv.at[j]], ssem[b]).wait()

    for b in range(NBUF):
      gather(b, b)

    def body(g, carry):
      for b in range(NBUF):
        j = g * NBUF + b
        gather_wait(j, b)
        scatter(j, b)
        scatter_wait(j, b)
        gather(j + NBUF, b)
      return carry

    lax.fori_loop(0, NGRP, body, 0)
    for b in range(NBUF):
      j = NGRP * NBUF + b
      gather_wait(j, b)
      scatter(j, b)
      scatter_wait(j, b)

    plsc.subcore_barrier()
    pltpu.async_copy(acc.at[my_rows], out.at[my_rows], sem).wait()
    plsc.subcore_barrier()

  @pl.when(cid == 0)
  def _():
    one_view(src1, dst1, t1b, s1a)
    one_view(src2, dst2, t2b, s2a)

  @pl.when(cid == 1)
  def _():
    one_view(src1, dst1, t1b, s1b_)
    one_view(src2, dst2, t2b, s2b_)


def _make_spmm_kernel():
  part = jax.ShapeDtypeStruct((N_ACC, D), jnp.int16)
  return functools.partial(
      pl.kernel,
      mesh=_sc_mesh(),
      out_type=(part, part, part, part),
      compiler_params=pltpu.CompilerParams(use_tc_tiling_on_sc=False),
      scratch_types=[
          pltpu.VMEM((EPT, LANES), jnp.int32),
          pltpu.VMEM((EPT, LANES), jnp.int32),
          pltpu.VMEM((NBUF, LANES, D), jnp.int16),
          pltpu.VMEM_SHARED((N_ACC, D), jnp.int16),
      ] + [pltpu.SemaphoreType.DMA] * (1 + 2 * NBUF),
  )(_spmm_body)


def _dinv(deg_col):
  return lax.rsqrt(jnp.maximum(deg_col + 1.0, 1e-12))


CAP = 1024.0


def _store_t(f32_ref, q_ref, scale_ref, t):
  s = CAP / jnp.maximum(jnp.max(jnp.abs(t)), 1e-30)
  scale_ref[...] = jnp.full((1, 1), s, jnp.float32)
  f32_ref[pl.ds(0, N), :] = t
  f32_ref[pl.ds(N, 16), :] = jnp.zeros((16, D), jnp.float32)
  q_ref[pl.ds(0, N), :] = jnp.round(t * s).astype(jnp.int16)
  q_ref[pl.ds(N, 16), :] = jnp.zeros((16, D), jnp.int16)


def _tc_pre_body(x1_ref, x2_ref, deg1_ref, deg2_ref, w_ref,
                 f1_ref, q1_ref, s1_ref, f2_ref, q2_ref, s2_ref):
  w = w_ref[...]
  t1 = _dinv(deg1_ref[...]) * jnp.dot(x1_ref[...], w,
                                      preferred_element_type=jnp.float32)
  _store_t(f1_ref, q1_ref, s1_ref, t1)
  t2 = _dinv(deg2_ref[...]) * jnp.dot(x2_ref[...], w,
                                      preferred_element_type=jnp.float32)
  _store_t(f2_ref, q2_ref, s2_ref, t2)


def _tc_pre(x1, x2, deg1, deg2, w):
  per_view = (jax.ShapeDtypeStruct((N_ACC, D), jnp.float32),
              jax.ShapeDtypeStruct((N_ACC, D), jnp.int16),
              jax.ShapeDtypeStruct((1, 1), jnp.float32))
  return pl.pallas_call(
      _tc_pre_body,
      out_shape=per_view + per_view,
  )(x1, x2, deg1, deg2, w)


def _combine(sa_ref, sb_ref, scale_ref):
  return (sa_ref[pl.ds(0, N), :].astype(jnp.float32) +
          sb_ref[pl.ds(0, N), :].astype(jnp.float32)) * (1.0 / scale_ref[...])


def _tc_mid_body(s1a_ref, s1b_ref, s2a_ref, s2b_ref, t1_ref, t2_ref,
                 sc1_ref, sc2_ref, deg1_ref, deg2_ref, b_ref, w_ref,
                 f1_ref, q1_ref, so1_ref, f2_ref, q2_ref, so2_ref):
  w = w_ref[...]
  b = b_ref[...]

  def one(sa_ref, sb_ref, t_ref, sc_ref, deg_ref, f_ref, q_ref, so_ref):
    dinv = _dinv(deg_ref[...])
    h = jax.nn.relu(
        dinv * (_combine(sa_ref, sb_ref, sc_ref) + t_ref[pl.ds(0, N), :]) + b)
    t2 = dinv * jnp.dot(h, w, preferred_element_type=jnp.float32)
    _store_t(f_ref, q_ref, so_ref, t2)

  one(s1a_ref, s1b_ref, t1_ref, sc1_ref, deg1_ref, f1_ref, q1_ref, so1_ref)
  one(s2a_ref, s2b_ref, t2_ref, sc2_ref, deg2_ref, f2_ref, q2_ref, so2_ref)


def _tc_mid(s1a, s1b, s2a, s2b, t1, t2, sc1, sc2, deg1, deg2, b, w):
  per_view = (jax.ShapeDtypeStruct((N_ACC, D), jnp.float32),
              jax.ShapeDtypeStruct((N_ACC, D), jnp.int16),
              jax.ShapeDtypeStruct((1, 1), jnp.float32))
  return pl.pallas_call(
      _tc_mid_body,
      out_shape=per_view + per_view,
  )(s1a, s1b, s2a, s2b, t1, t2, sc1, sc2, deg1, deg2, b, w)


def _tc_fin_body(s1a_ref, s1b_ref, s2a_ref, s2b_ref, t1_ref, t2_ref,
                 sc1_ref, sc2_ref, deg1_ref, deg2_ref, b_ref,
                 z1_ref, z2_ref):
  b = b_ref[...]

  def one(sa_ref, sb_ref, t_ref, sc_ref, deg_ref, out_ref):
    dinv = _dinv(deg_ref[...])
    h = (dinv * (_combine(sa_ref, sb_ref, sc_ref) + t_ref[pl.ds(0, N), :])
         + b)
    mu = jnp.sum(h, axis=0, keepdims=True) * (1.0 / N)
    d = h - mu
    var = jnp.sum(d * d, axis=0, keepdims=True) * (1.0 / (N - 1))
    out_ref[...] = d / jnp.sqrt(var)

  one(s1a_ref, s1b_ref, t1_ref, sc1_ref, deg1_ref, z1_ref)
  one(s2a_ref, s2b_ref, t2_ref, sc2_ref, deg2_ref, z2_ref)


def _tc_fin(s1a, s1b, s2a, s2b, t1, t2, sc1, sc2, deg1, deg2, b):
  z = jax.ShapeDtypeStruct((N, D), jnp.float32)
  return pl.pallas_call(
      _tc_fin_body,
      out_shape=(z, z),
  )(s1a, s1b, s2a, s2b, t1, t2, sc1, sc2, deg1, deg2, b)


def _prep_edges(ei):
  pad = HALF_ROWS * LANES - EH
  fill = jnp.full((pad,), N, jnp.int32)

  def prep(row):
    row = row.astype(jnp.int32)
    return jnp.concatenate([row[:EH], fill, row[EH:], fill]).reshape(
        IDX_ROWS, LANES)

  return prep(ei[0]), prep(ei[1])


def kernel(edge_index_1, x_1, edge_index_2, x_2, W1, b1, W2, b2):
  src1, dst1 = _prep_edges(edge_index_1)
  src2, dst2 = _prep_edges(edge_index_2)
  zeros1d = jnp.zeros((ROWS_PER_TILE,), jnp.float32)
  zeros2d = jnp.zeros((ROWS_PER_TILE, D), jnp.int16)
  b1r = b1.reshape(1, D)
  b2r = b2.reshape(1, D)

  deg_kernel = _make_deg_kernel()
  spmm_kernel = _make_spmm_kernel()

  deg1, deg2 = deg_kernel(dst1, dst2, zeros1d)
  deg1c = deg1[:N].reshape(N, 1)
  deg2c = deg2[:N].reshape(N, 1)

  t1_1, t1_1q, sc1_1, t1_2, t1_2q, sc1_2 = _tc_pre(
      x_1, x_2, deg1c, deg2c, W1)

  s1_1a, s1_1b, s1_2a, s1_2b = spmm_kernel(
      src1, dst1, src2, dst2, t1_1q, t1_2q, zeros2d)

  t2_1, t2_1q, sc2_1, t2_2, t2_2q, sc2_2 = _tc_mid(
      s1_1a, s1_1b, s1_2a, s1_2b, t1_1, t1_2, sc1_1, sc1_2,
      deg1c, deg2c, b1r, W2)

  s2_1a, s2_1b, s2_2a, s2_2b = spmm_kernel(
      src1, dst1, src2, dst2, t2_1q, t2_2q, zeros2d)

  z1, z2 = _tc_fin(
      s2_1a, s2_1b, s2_2a, s2_2b, t2_1, t2_2, sc2_1, sc2_2,
      deg1c, deg2c, b2r)
  return (z1, z2)


# speedup vs baseline: 1.0327x; 1.0054x over previous
import functools

import jax
import jax.numpy as jnp
from jax import lax
from jax.experimental import pallas as pl
from jax.experimental.pallas import tpu as pltpu
from jax.experimental.pallas import tpu_sc as plsc

N = 10000
E = 320000
D = 128

NUM_TILES = 16
LANES = 128
EH = E // 2
EPT = 80
HALF_ROWS = EPT * NUM_TILES
IDX_ROWS = 2 * HALF_ROWS
EPT_DEG = IDX_ROWS // NUM_TILES
N_ACC = 10240
ROWS_PER_TILE = N_ACC // NUM_TILES


def _sc_mesh():
  return plsc.VectorSubcoreMesh(core_axis_name="c", subcore_axis_name="s")


def _deg_body(dst1, dst2, zeros1d, deg1, deg2, dstv, ones_v, acc, sem):
  cid = lax.axis_index("c")
  sid = lax.axis_index("s")
  for c in range(8):
    ones_v[pl.ds(c * 16, 16)] = jnp.ones((16,), jnp.float32)
  pltpu.async_copy(zeros1d, acc.at[pl.ds(sid * ROWS_PER_TILE, ROWS_PER_TILE)],
                   sem).wait()
  plsc.subcore_barrier()

  def run(dstm):
    pltpu.async_copy(dstm.at[pl.ds(sid * EPT_DEG, EPT_DEG)], dstv, sem).wait()

    def body(j, carry):
      pltpu.async_copy(ones_v, acc.at[dstv.at[j]], sem, add=True).wait()
      return carry

    lax.fori_loop(0, EPT_DEG, body, 0)

  @pl.when(cid == 0)
  def _():
    run(dst1)

  @pl.when(cid == 1)
  def _():
    run(dst2)

  plsc.subcore_barrier()

  def wb(out):
    pltpu.async_copy(acc.at[pl.ds(sid * ROWS_PER_TILE, ROWS_PER_TILE)],
                     out.at[pl.ds(sid * ROWS_PER_TILE, ROWS_PER_TILE)],
                     sem).wait()

  @pl.when(cid == 0)
  def _():
    wb(deg1)

  @pl.when(cid == 1)
  def _():
    wb(deg2)


def _make_deg_kernel():
  return functools.partial(
      pl.kernel,
      mesh=_sc_mesh(),
      out_type=(jax.ShapeDtypeStruct((N_ACC,), jnp.float32),
                jax.ShapeDtypeStruct((N_ACC,), jnp.float32)),
      scratch_types=[
          pltpu.VMEM((EPT_DEG, LANES), jnp.int32),
          pltpu.VMEM((LANES,), jnp.float32),
          pltpu.VMEM_SHARED((N_ACC,), jnp.float32),
          pltpu.SemaphoreType.DMA,
      ],
  )(_deg_body)


NBUF = 8
NGRP = EPT // NBUF - 1


def _spmm_body(src1, dst1, src2, dst2, t1b, t2b, zeros2d,
               s1a, s1b_, s2a, s2b_, srcv, dstv, rows, acc, *sems):
  cid = lax.axis_index("c")
  sid = lax.axis_index("s")
  my_rows = pl.ds(sid * ROWS_PER_TILE, ROWS_PER_TILE)
  sem = sems[0]
  gsem = sems[1:1 + NBUF]
  ssem = sems[1 + NBUF:1 + 2 * NBUF]
  idx_base = cid * HALF_ROWS + sid * EPT

  def one_view(srcm, dstm, t_hbm, out):
    pltpu.async_copy(zeros2d, acc.at[my_rows], sem).wait()
    pltpu.async_copy(srcm.at[pl.ds(idx_base, EPT)], srcv, sem).wait()
    pltpu.async_copy(dstm.at[pl.ds(idx_base, EPT)], dstv, sem).wait()
    plsc.subcore_barrier()

    def gather(j, b):
      pltpu.make_async_copy(t_hbm.at[srcv.at[j]], rows.at[b], gsem[b]).start()

    def gather_wait(j, b):
      pltpu.make_async_copy(t_hbm.at[srcv.at[j]], rows.at[b], gsem[b]).wait()

    def scatter(j, b):
      pltpu.make_async_copy(rows.at[b], acc.at[dstv.at[j]], ssem[b]).start(
          add=True)

    def scatter_wait(j, b):
      pltpu.make_async_copy(rows.at[b], acc.at[dstv.at[j]], ssem[b]).wait()

    for b in range(NBUF):
      gather(b, b)

    def body(g, carry):
      for b in range(NBUF):
        j = g * NBUF + b
        gather_wait(j, b)
        scatter(j, b)
        scatter_wait(j, b)
        gather(j + NBUF, b)
      return carry

    lax.fori_loop(0, NGRP, body, 0)
    for b in range(NBUF):
      j = NGRP * NBUF + b
      gather_wait(j, b)
      scatter(j, b)
      scatter_wait(j, b)

    plsc.subcore_barrier()
    pltpu.async_copy(acc.at[my_rows], out.at[my_rows], sem).wait()
    plsc.subcore_barrier()

  @pl.when(cid == 0)
  def _():
    one_view(src1, dst1, t1b, s1a)
    one_view(src2, dst2, t2b, s2a)

  @pl.when(cid == 1)
  def _():
    one_view(src1, dst1, t1b, s1b_)
    one_view(src2, dst2, t2b, s2b_)


def _make_spmm_kernel():
  part = jax.ShapeDtypeStruct((N_ACC, D), jnp.int16)
  return functools.partial(
      pl.kernel,
      mesh=_sc_mesh(),
      out_type=(part, part, part, part),
      compiler_params=pltpu.CompilerParams(use_tc_tiling_on_sc=False),
      scratch_types=[
          pltpu.VMEM((EPT, LANES), jnp.int32),
          pltpu.VMEM((EPT, LANES), jnp.int32),
          pltpu.VMEM((NBUF, LANES, D), jnp.int16),
          pltpu.VMEM_SHARED((N_ACC, D), jnp.int16),
      ] + [pltpu.SemaphoreType.DMA] * (1 + 2 * NBUF),
  )(_spmm_body)


def _dinv(deg_col):
  return lax.rsqrt(jnp.maximum(deg_col + 1.0, 1e-12))


CAP = 1024.0


def _store_t(f32_ref, q_ref, scale_ref, t):
  s = CAP / jnp.maximum(jnp.max(jnp.abs(t)), 1e-30)
  scale_ref[...] = jnp.full((1, 1), s, jnp.float32)
  f32_ref[pl.ds(0, N), :] = t
  f32_ref[pl.ds(N, 16), :] = jnp.zeros((16, D), jnp.float32)
  q_ref[pl.ds(0, N), :] = jnp.round(t * s).astype(jnp.int16)
  q_ref[pl.ds(N, 16), :] = jnp.zeros((16, D), jnp.int16)


def _tc_pre_body(x1_ref, x2_ref, deg1_ref, deg2_ref, w_ref,
                 f1_ref, q1_ref, s1_ref, f2_ref, q2_ref, s2_ref):
  w = w_ref[...]
  t1 = _dinv(deg1_ref[...]) * jnp.dot(x1_ref[...], w,
                                      preferred_element_type=jnp.float32)
  _store_t(f1_ref, q1_ref, s1_ref, t1)
  t2 = _dinv(deg2_ref[...]) * jnp.dot(x2_ref[...], w,
                                      preferred_element_type=jnp.float32)
  _store_t(f2_ref, q2_ref, s2_ref, t2)


def _tc_pre(x1, x2, deg1, deg2, w):
  per_view = (jax.ShapeDtypeStruct((N_ACC, D), jnp.float32),
              jax.ShapeDtypeStruct((N_ACC, D), jnp.int16),
              jax.ShapeDtypeStruct((1, 1), jnp.float32))
  return pl.pallas_call(
      _tc_pre_body,
      out_shape=per_view + per_view,
  )(x1, x2, deg1, deg2, w)


def _combine(sa_ref, sb_ref, scale_ref):
  return (sa_ref[pl.ds(0, N), :].astype(jnp.float32) +
          sb_ref[pl.ds(0, N), :].astype(jnp.float32)) * (1.0 / scale_ref[...])


def _tc_mid_body(s1a_ref, s1b_ref, s2a_ref, s2b_ref, t1_ref, t2_ref,
                 sc1_ref, sc2_ref, deg1_ref, deg2_ref, b_ref, w_ref,
                 f1_ref, q1_ref, so1_ref, f2_ref, q2_ref, so2_ref):
  w = w_ref[...]
  b = b_ref[...]

  def one(sa_ref, sb_ref, t_ref, sc_ref, deg_ref, f_ref, q_ref, so_ref):
    dinv = _dinv(deg_ref[...])
    h = jax.nn.relu(
        dinv * (_combine(sa_ref, sb_ref, sc_ref) + t_ref[pl.ds(0, N), :]) + b)
    t2 = dinv * jnp.dot(h, w, preferred_element_type=jnp.float32)
    _store_t(f_ref, q_ref, so_ref, t2)

  one(s1a_ref, s1b_ref, t1_ref, sc1_ref, deg1_ref, f1_ref, q1_ref, so1_ref)
  one(s2a_ref, s2b_ref, t2_ref, sc2_ref, deg2_ref, f2_ref, q2_ref, so2_ref)


def _tc_mid(s1a, s1b, s2a, s2b, t1, t2, sc1, sc2, deg1, deg2, b, w):
  per_view = (jax.ShapeDtypeStruct((N_ACC, D), jnp.float32),
              jax.ShapeDtypeStruct((N_ACC, D), jnp.int16),
              jax.ShapeDtypeStruct((1, 1), jnp.float32))
  return pl.pallas_call(
      _tc_mid_body,
      out_shape=per_view + per_view,
  )(s1a, s1b, s2a, s2b, t1, t2, sc1, sc2, deg1, deg2, b, w)


def _tc_fin_body(s1a_ref, s1b_ref, s2a_ref, s2b_ref, t1_ref, t2_ref,
                 sc1_ref, sc2_ref, deg1_ref, deg2_ref, b_ref,
                 z1_ref, z2_ref):
  b = b_ref[...]

  def one(sa_ref, sb_ref, t_ref, sc_ref, deg_ref, out_ref):
    dinv = _dinv(deg_ref[...])
    h = (dinv * (_combine(sa_ref, sb_ref, sc_ref) + t_ref[pl.ds(0, N), :])
         + b)
    mu = jnp.sum(h, axis=0, keepdims=True) * (1.0 / N)
    d = h - mu
    var = jnp.sum(d * d, axis=0, keepdims=True) * (1.0 / (N - 1))
    out_ref[...] = d / jnp.sqrt(var)

  one(s1a_ref, s1b_ref, t1_ref, sc1_ref, deg1_ref, z1_ref)
  one(s2a_ref, s2b_ref, t2_ref, sc2_ref, deg2_ref, z2_ref)


def _tc_fin(s1a, s1b, s2a, s2b, t1, t2, sc1, sc2, deg1, deg2, b):
  z = jax.ShapeDtypeStruct((N, D), jnp.float32)
  return pl.pallas_call(
      _tc_fin_body,
      out_shape=(z, z),
  )(s1a, s1b, s2a, s2b, t1, t2, sc1, sc2, deg1, deg2, b)


def _prep_edges(ei):
  pad = HALF_ROWS * LANES - EH
  fill = jnp.full((pad,), N, jnp.int32)

  def prep(row):
    row = row.astype(jnp.int32)
    return jnp.concatenate([row[:EH], fill, row[EH:], fill]).reshape(
        IDX_ROWS, LANES)

  return prep(ei[0]), prep(ei[1])


def kernel(edge_index_1, x_1, edge_index_2, x_2, W1, b1, W2, b2):
  src1, dst1 = _prep_edges(edge_index_1)
  src2, dst2 = _prep_edges(edge_index_2)
  zeros1d = jnp.zeros((ROWS_PER_TILE,), jnp.float32)
  zeros2d = jnp.zeros((ROWS_PER_TILE, D), jnp.int16)
  b1r = b1.reshape(1, D)
  b2r = b2.reshape(1, D)

  deg_kernel = _make_deg_kernel()
  spmm_kernel = _make_spmm_kernel()

  deg1, deg2 = deg_kernel(dst1, dst2, zeros1d)
  deg1c = deg1[:N].reshape(N, 1)
  deg2c = deg2[:N].reshape(N, 1)

  t1_1, t1_1q, sc1_1, t1_2, t1_2q, sc1_2 = _tc_pre(
      x_1, x_2, deg1c, deg2c, W1)

  s1_1a, s1_1b, s1_2a, s1_2b = spmm_kernel(
      src1, dst1, src2, dst2, t1_1q, t1_2q, zeros2d)

  t2_1, t2_1q, sc2_1, t2_2, t2_2q, sc2_2 = _tc_mid(
      s1_1a, s1_1b, s1_2a, s1_2b, t1_1, t1_2, sc1_1, sc1_2,
      deg1c, deg2c, b1r, W2)

  s2_1a, s2_1b, s2_2a, s2_2b = spmm_kernel(
      src1, dst1, src2, dst2, t2_1q, t2_2q, zeros2d)

  z1, z2 = _tc_fin(
      s2_1a, s2_1b, s2_2a, s2_2b, t2_1, t2_2, sc2_1, sc2_2,
      deg1c, deg2c, b2r)
  return (z1, z2)
